# Initial kernel scaffold; baseline (speedup 1.0000x reference)
import jax
import jax.numpy as jnp
from jax.experimental import pallas as pl


def kernel(idx0_batch, idx1_batch, emb0_w, emb1_w, fc2_w, fc2_b, fc3_w, fc3_b, fc4_w, fc4_b):
    raise NotImplementedError("write your pallas kernel here")


# speedup vs baseline: 17.9549x; 17.9549x over previous
import jax
import jax.numpy as jnp
from jax.experimental import pallas as pl
from jax.experimental.pallas import tpu as pltpu

TABLE_SIZE = 640
HIDDEN = 256
B = 4096
L = 50
LPAD = 56
BT = 512


def _fwd_kernel(idx0_ref, idx1_ref, emb0_ref, emb1_ref, w2_ref, b2_ref,
                w3_ref, b3_ref, w4_ref, b4_ref, out_ref):
    iota = jax.lax.broadcasted_iota(jnp.int32, (TABLE_SIZE, BT), 0)
    c0 = jnp.zeros((TABLE_SIZE, BT), jnp.float32)
    c1 = jnp.zeros((TABLE_SIZE, BT), jnp.float32)
    for l in range(LPAD):
        c0 = c0 + (idx0_ref[l, :][None, :] == iota).astype(jnp.float32)
        c1 = c1 + (idx1_ref[l, :][None, :] == iota).astype(jnp.float32)
    dn_tt = (((0,), (0,)), ((), ()))
    sum0 = jax.lax.dot_general(c0, emb0_ref[...], dn_tt,
                               preferred_element_type=jnp.float32)
    sum1 = jax.lax.dot_general(c1, emb1_ref[...], dn_tt,
                               preferred_element_type=jnp.float32)
    h0 = jnp.maximum(sum0, 0.0)
    h1 = jnp.maximum(sum1, 0.0)
    dn_nt = (((1,), (1,)), ((), ()))
    w2 = w2_ref[...]
    x = (jax.lax.dot_general(h0, w2[:, :HIDDEN], dn_nt,
                             preferred_element_type=jnp.float32)
         + jax.lax.dot_general(h1, w2[:, HIDDEN:], dn_nt,
                               preferred_element_type=jnp.float32)
         + b2_ref[...])
    x = jnp.maximum(x, 0.0)
    x = jax.lax.dot_general(x, w3_ref[...], dn_nt,
                            preferred_element_type=jnp.float32) + b3_ref[...]
    x = jnp.maximum(x, 0.0)
    out = jnp.sum(x * w4_ref[...], axis=1, keepdims=True) + b4_ref[...]
    out_ref[...] = out


@jax.jit
def kernel(idx0_batch, idx1_batch, emb0_w, emb1_w, fc2_w, fc2_b, fc3_w,
           fc3_b, fc4_w, fc4_b):
    pad = ((0, LPAD - L), (0, 0))
    idx0 = jnp.pad(idx0_batch.astype(jnp.int32).T, pad,
                   constant_values=TABLE_SIZE)
    idx1 = jnp.pad(idx1_batch.astype(jnp.int32).T, pad,
                   constant_values=TABLE_SIZE)
    b2 = fc2_b.reshape(1, -1)
    b3 = fc3_b.reshape(1, -1)
    b4 = fc4_b.reshape(1, 1)
    grid = (B // BT,)
    out = pl.pallas_call(
        _fwd_kernel,
        grid=grid,
        in_specs=[
            pl.BlockSpec((LPAD, BT), lambda i: (0, i)),
            pl.BlockSpec((LPAD, BT), lambda i: (0, i)),
            pl.BlockSpec((TABLE_SIZE, HIDDEN), lambda i: (0, 0)),
            pl.BlockSpec((TABLE_SIZE, HIDDEN), lambda i: (0, 0)),
            pl.BlockSpec(fc2_w.shape, lambda i: (0, 0)),
            pl.BlockSpec(b2.shape, lambda i: (0, 0)),
            pl.BlockSpec(fc3_w.shape, lambda i: (0, 0)),
            pl.BlockSpec(b3.shape, lambda i: (0, 0)),
            pl.BlockSpec(fc4_w.shape, lambda i: (0, 0)),
            pl.BlockSpec(b4.shape, lambda i: (0, 0)),
        ],
        out_specs=pl.BlockSpec((BT, 1), lambda i: (i, 0)),
        out_shape=jax.ShapeDtypeStruct((B, 1), jnp.float32),
        compiler_params=pltpu.CompilerParams(
            dimension_semantics=("arbitrary",),
        ),
    )(idx0, idx1, emb0_w, emb1_w, fc2_w, b2, fc3_w, b3, fc4_w, b4)
    return out[:, 0]


# speedup vs baseline: 27.9306x; 1.5556x over previous
import functools

import jax
import jax.numpy as jnp
from jax import lax
from jax.experimental import pallas as pl
from jax.experimental.pallas import tpu as pltpu
from jax.experimental.pallas import tpu_sc as plsc

TABLE_SIZE = 640
HIDDEN = 256
B = 4096
L = 50
CW = 2 * TABLE_SIZE
NL = 2 * L

_info = plsc.get_sparse_core_info()
NC, NS = _info.num_cores, _info.num_subcores
NW = NC * NS
SPT = B // NW
HALF = SPT // 2
NIDX = HALF * NL
HWORDS = HALF * CW

_mesh = plsc.VectorSubcoreMesh(core_axis_name="c", subcore_axis_name="s")


@functools.partial(
    pl.kernel,
    mesh=_mesh,
    out_type=jax.ShapeDtypeStruct((B * CW,), jnp.float32),
    scratch_types=[
        pltpu.VMEM((NIDX,), jnp.int32),
        pltpu.VMEM((HWORDS,), jnp.float32),
        pltpu.SemaphoreType.DMA,
    ],
    compiler_params=pltpu.CompilerParams(needs_layout_passes=False),
)
def _sc_hist(addr_hbm, out_hbm, addr_v, counts_v, sem):
    wid = lax.axis_index("s") * NC + lax.axis_index("c")
    zero16 = jnp.zeros((16,), jnp.float32)
    ones16 = jnp.full((16,), 1.0, jnp.float32)
    for p in range(2):
        base = wid * SPT + p * HALF
        pltpu.sync_copy(addr_hbm.at[pl.ds(base * NL, NIDX)], addr_v)

        def zbody(i, carry):
            w = i * 256
            for k in range(16):
                counts_v[pl.ds(w + k * 16, 16)] = zero16
            return carry

        lax.fori_loop(0, HWORDS // 256, zbody, 0)

        def sbody(i, carry):
            w = i * 64
            for k in range(4):
                a = addr_v[pl.ds(w + k * 16, 16)]
                plsc.addupdate_scatter(counts_v, [a], ones16)
            return carry

        lax.fori_loop(0, NIDX // 64, sbody, 0)

        pltpu.sync_copy(counts_v, out_hbm.at[pl.ds(base * CW, HWORDS)])


BT = 512


def _tc_kernel(cnt_ref, emb0_ref, emb1_ref, w2_ref, b2_ref,
               w3_ref, b3_ref, w4_ref, b4_ref, out_ref):
    c = cnt_ref[...]
    dn = (((1,), (0,)), ((), ()))
    sum0 = jax.lax.dot_general(c[:, :TABLE_SIZE], emb0_ref[...], dn,
                               preferred_element_type=jnp.float32)
    sum1 = jax.lax.dot_general(c[:, TABLE_SIZE:], emb1_ref[...], dn,
                               preferred_element_type=jnp.float32)
    h0 = jnp.maximum(sum0, 0.0)
    h1 = jnp.maximum(sum1, 0.0)
    dn_nt = (((1,), (1,)), ((), ()))
    w2 = w2_ref[...]
    x = (jax.lax.dot_general(h0, w2[:, :HIDDEN], dn_nt,
                             preferred_element_type=jnp.float32)
         + jax.lax.dot_general(h1, w2[:, HIDDEN:], dn_nt,
                               preferred_element_type=jnp.float32)
         + b2_ref[...])
    x = jnp.maximum(x, 0.0)
    x = jax.lax.dot_general(x, w3_ref[...], dn_nt,
                            preferred_element_type=jnp.float32) + b3_ref[...]
    x = jnp.maximum(x, 0.0)
    out = jnp.sum(x * w4_ref[...], axis=1, keepdims=True) + b4_ref[...]
    out_ref[...] = out


@jax.jit
def kernel(idx0_batch, idx1_batch, emb0_w, emb1_w, fc2_w, fc2_b, fc3_w,
           fc3_b, fc4_w, fc4_b):
    idx0 = idx0_batch.astype(jnp.int32)
    idx1 = idx1_batch.astype(jnp.int32)
    local = (jnp.arange(B, dtype=jnp.int32)[:, None] % HALF) * CW
    addr = jnp.concatenate([idx0, idx1 + TABLE_SIZE], axis=1) + local
    counts_flat = _sc_hist(addr.reshape(-1))
    counts = counts_flat.reshape(B, CW)

    b2 = fc2_b.reshape(1, -1)
    b3 = fc3_b.reshape(1, -1)
    b4 = fc4_b.reshape(1, 1)
    out = pl.pallas_call(
        _tc_kernel,
        grid=(B // BT,),
        in_specs=[
            pl.BlockSpec((BT, CW), lambda i: (i, 0)),
            pl.BlockSpec((TABLE_SIZE, HIDDEN), lambda i: (0, 0)),
            pl.BlockSpec((TABLE_SIZE, HIDDEN), lambda i: (0, 0)),
            pl.BlockSpec(fc2_w.shape, lambda i: (0, 0)),
            pl.BlockSpec(b2.shape, lambda i: (0, 0)),
            pl.BlockSpec(fc3_w.shape, lambda i: (0, 0)),
            pl.BlockSpec(b3.shape, lambda i: (0, 0)),
            pl.BlockSpec(fc4_w.shape, lambda i: (0, 0)),
            pl.BlockSpec(b4.shape, lambda i: (0, 0)),
        ],
        out_specs=pl.BlockSpec((BT, 1), lambda i: (i, 0)),
        out_shape=jax.ShapeDtypeStruct((B, 1), jnp.float32),
        compiler_params=pltpu.CompilerParams(
            dimension_semantics=("arbitrary",),
        ),
    )(counts, emb0_w, emb1_w, fc2_w, b2, fc3_w, b3, fc4_w, b4)
    return out[:, 0]


# speedup vs baseline: 37.3359x; 1.3367x over previous
import functools

import jax
import jax.numpy as jnp
from jax import lax
from jax.experimental import pallas as pl
from jax.experimental.pallas import tpu as pltpu
from jax.experimental.pallas import tpu_sc as plsc

TABLE_SIZE = 640
HIDDEN = 256
B = 4096
L = 50
CW = 2 * TABLE_SIZE
NL = 2 * L
NLP = 128
NV = NL // 16
NREM = NL - NV * 16

_info = plsc.get_sparse_core_info()
NC, NS = _info.num_cores, _info.num_subcores
NW = NC * NS
SPT = B // NW
QS = 32
NQ = SPT // QS

_mesh = plsc.VectorSubcoreMesh(core_axis_name="c", subcore_axis_name="s")


@functools.partial(
    pl.kernel,
    mesh=_mesh,
    out_type=jax.ShapeDtypeStruct((B, CW), jnp.float32),
    scratch_types=[
        pltpu.VMEM((QS, NLP), jnp.int32),
        pltpu.VMEM((QS, CW), jnp.float32),
        pltpu.VMEM((QS, CW), jnp.float32),
        pltpu.SemaphoreType.DMA,
        pltpu.SemaphoreType.DMA,
    ],
    compiler_params=pltpu.CompilerParams(needs_layout_passes=False),
)
def _sc_hist(cols_hbm, out_hbm, cols_v, cnt_a, cnt_b, sem_a, sem_b):
    wid = lax.axis_index("s") * NC + lax.axis_index("c")
    zero16 = jnp.zeros((16,), jnp.float32)
    ones16 = jnp.full((16,), 1.0, jnp.float32)
    rem_mask = lax.iota(jnp.int32, 16) < NREM
    bufs = (cnt_a, cnt_b)
    sems = (sem_a, sem_b)
    pending = [None, None]
    for q in range(NQ):
        buf = bufs[q % 2]
        sem = sems[q % 2]
        if pending[q % 2] is not None:
            pending[q % 2].wait()
        base = wid * SPT + q * QS
        pltpu.sync_copy(cols_hbm.at[pl.ds(base, QS)], cols_v)

        def row_body(s, carry, buf=buf):
            for k in range(80):
                buf[s, pl.ds(k * 16, 16)] = zero16
            row16 = jnp.full((16,), s, jnp.int32)
            for k in range(NV):
                cv = cols_v[s, pl.ds(k * 16, 16)]
                plsc.addupdate_scatter(buf, [row16, cv], ones16)
            cv = cols_v[s, pl.ds(NV * 16, 16)]
            plsc.addupdate_scatter(buf, [row16, cv], ones16, mask=rem_mask)
            return carry

        lax.fori_loop(0, QS, row_body, 0)
        pending[q % 2] = pltpu.async_copy(
            buf, out_hbm.at[pl.ds(base, QS)], sem)
    pending[0].wait()
    pending[1].wait()


BT = 512


def _tc_kernel(cnt_ref, emb0_ref, emb1_ref, w2_ref, b2_ref,
               w3_ref, b3_ref, w4_ref, b4_ref, out_ref):
    c = cnt_ref[...]
    dn = (((1,), (0,)), ((), ()))
    sum0 = jax.lax.dot_general(c[:, :TABLE_SIZE], emb0_ref[...], dn,
                               preferred_element_type=jnp.float32)
    sum1 = jax.lax.dot_general(c[:, TABLE_SIZE:], emb1_ref[...], dn,
                               preferred_element_type=jnp.float32)
    h0 = jnp.maximum(sum0, 0.0)
    h1 = jnp.maximum(sum1, 0.0)
    dn_nt = (((1,), (1,)), ((), ()))
    w2 = w2_ref[...]
    x = (jax.lax.dot_general(h0, w2[:, :HIDDEN], dn_nt,
                             preferred_element_type=jnp.float32)
         + jax.lax.dot_general(h1, w2[:, HIDDEN:], dn_nt,
                               preferred_element_type=jnp.float32)
         + b2_ref[...])
    x = jnp.maximum(x, 0.0)
    x = jax.lax.dot_general(x, w3_ref[...], dn_nt,
                            preferred_element_type=jnp.float32) + b3_ref[...]
    x = jnp.maximum(x, 0.0)
    out = jnp.sum(x * w4_ref[...], axis=1, keepdims=True) + b4_ref[...]
    out_ref[...] = out


@jax.jit
def kernel(idx0_batch, idx1_batch, emb0_w, emb1_w, fc2_w, fc2_b, fc3_w,
           fc3_b, fc4_w, fc4_b):
    idx0 = idx0_batch.astype(jnp.int32)
    idx1 = idx1_batch.astype(jnp.int32)
    cols = jnp.concatenate(
        [idx0, idx1 + TABLE_SIZE,
         jnp.zeros((B, NLP - NL), jnp.int32)], axis=1)
    counts = _sc_hist(cols)

    b2 = fc2_b.reshape(1, -1)
    b3 = fc3_b.reshape(1, -1)
    b4 = fc4_b.reshape(1, 1)
    out = pl.pallas_call(
        _tc_kernel,
        grid=(B // BT,),
        in_specs=[
            pl.BlockSpec((BT, CW), lambda i: (i, 0)),
            pl.BlockSpec((TABLE_SIZE, HIDDEN), lambda i: (0, 0)),
            pl.BlockSpec((TABLE_SIZE, HIDDEN), lambda i: (0, 0)),
            pl.BlockSpec(fc2_w.shape, lambda i: (0, 0)),
            pl.BlockSpec(b2.shape, lambda i: (0, 0)),
            pl.BlockSpec(fc3_w.shape, lambda i: (0, 0)),
            pl.BlockSpec(b3.shape, lambda i: (0, 0)),
            pl.BlockSpec(fc4_w.shape, lambda i: (0, 0)),
            pl.BlockSpec(b4.shape, lambda i: (0, 0)),
        ],
        out_specs=pl.BlockSpec((BT, 1), lambda i: (i, 0)),
        out_shape=jax.ShapeDtypeStruct((B, 1), jnp.float32),
        compiler_params=pltpu.CompilerParams(
            dimension_semantics=("arbitrary",),
        ),
    )(counts, emb0_w, emb1_w, fc2_w, b2, fc3_w, b3, fc4_w, b4)
    return out[:, 0]


# speedup vs baseline: 40.4219x; 1.0827x over previous
import functools

import jax
import jax.numpy as jnp
from jax import lax
from jax.experimental import pallas as pl
from jax.experimental.pallas import tpu as pltpu
from jax.experimental.pallas import tpu_sc as plsc

TABLE_SIZE = 640
HIDDEN = 256
B = 4096
L = 50
CW = 2 * TABLE_SIZE

_info = plsc.get_sparse_core_info()
NC, NS = _info.num_cores, _info.num_subcores
NW = NC * NS
SPT = B // NW
QS = 32
NQ = SPT // QS

_mesh = plsc.VectorSubcoreMesh(core_axis_name="c", subcore_axis_name="s")


@functools.partial(
    pl.kernel,
    mesh=_mesh,
    out_type=jax.ShapeDtypeStruct((B, CW), jnp.float32),
    scratch_types=[
        pltpu.VMEM((SPT, L), jnp.int32),
        pltpu.VMEM((SPT, L), jnp.int32),
        pltpu.VMEM((QS, CW), jnp.float32),
        pltpu.VMEM((QS, CW), jnp.float32),
        pltpu.SemaphoreType.DMA,
        pltpu.SemaphoreType.DMA,
        pltpu.SemaphoreType.DMA,
    ],
    compiler_params=pltpu.CompilerParams(needs_layout_passes=False),
)
def _sc_hist(idx0_hbm, idx1_hbm, out_hbm, idx0_v, idx1_v, cnt_a, cnt_b,
             sem_a, sem_b, sem_i):
    wid = lax.axis_index("s") * NC + lax.axis_index("c")
    sbase = wid * SPT
    zero16 = jnp.zeros((16,), jnp.float32)
    ones16 = jnp.full((16,), 1.0, jnp.float32)
    rem_mask = lax.iota(jnp.int32, 16) >= 14
    off1 = jnp.full((16,), TABLE_SIZE, jnp.int32)
    ld0 = pltpu.async_copy(idx0_hbm.at[pl.ds(sbase, SPT)], idx0_v, sem_i)
    ld1 = pltpu.async_copy(idx1_hbm.at[pl.ds(sbase, SPT)], idx1_v, sem_i)
    bufs = (cnt_a, cnt_b)
    sems = (sem_a, sem_b)
    pending = [None, None]
    for q in range(NQ):
        buf = bufs[q % 2]
        sem = sems[q % 2]
        if pending[q % 2] is not None:
            pending[q % 2].wait()

        def zrow_body(s, carry, buf=buf):
            for k in range(80):
                buf[s, pl.ds(k * 16, 16)] = zero16
            return carry

        lax.fori_loop(0, QS, zrow_body, 0)
        if q == 0:
            ld0.wait()
            ld1.wait()

        def srow_body(s, carry, buf=buf, q=q):
            r = q * QS + s
            row16 = jnp.full((16,), s, jnp.int32)
            for k in range(3):
                cv = idx0_v[r, pl.ds(k * 16, 16)]
                plsc.addupdate_scatter(buf, [row16, cv], ones16)
            cv = idx0_v[r, pl.ds(34, 16)]
            plsc.addupdate_scatter(buf, [row16, cv], ones16, mask=rem_mask)
            for k in range(3):
                cv = idx1_v[r, pl.ds(k * 16, 16)] + off1
                plsc.addupdate_scatter(buf, [row16, cv], ones16)
            cv = idx1_v[r, pl.ds(34, 16)] + off1
            plsc.addupdate_scatter(buf, [row16, cv], ones16, mask=rem_mask)
            return carry

        lax.fori_loop(0, QS, srow_body, 0)
        pending[q % 2] = pltpu.async_copy(
            buf, out_hbm.at[pl.ds(sbase + q * QS, QS)], sem)
    pending[0].wait()
    pending[1].wait()


BT = 512


def _tc_kernel(cnt_ref, emb0_ref, emb1_ref, w2_ref, b2_ref,
               w3_ref, b3_ref, w4_ref, b4_ref, out_ref):
    c = cnt_ref[...]
    dn = (((1,), (0,)), ((), ()))
    sum0 = jax.lax.dot_general(c[:, :TABLE_SIZE], emb0_ref[...], dn,
                               preferred_element_type=jnp.float32)
    sum1 = jax.lax.dot_general(c[:, TABLE_SIZE:], emb1_ref[...], dn,
                               preferred_element_type=jnp.float32)
    h0 = jnp.maximum(sum0, 0.0)
    h1 = jnp.maximum(sum1, 0.0)
    dn_nt = (((1,), (1,)), ((), ()))
    w2 = w2_ref[...]
    x = (jax.lax.dot_general(h0, w2[:, :HIDDEN], dn_nt,
                             preferred_element_type=jnp.float32)
         + jax.lax.dot_general(h1, w2[:, HIDDEN:], dn_nt,
                               preferred_element_type=jnp.float32)
         + b2_ref[...])
    x = jnp.maximum(x, 0.0)
    x = jax.lax.dot_general(x, w3_ref[...], dn_nt,
                            preferred_element_type=jnp.float32) + b3_ref[...]
    x = jnp.maximum(x, 0.0)
    out = jnp.sum(x * w4_ref[...], axis=1, keepdims=True) + b4_ref[...]
    out_ref[...] = out


@jax.jit
def kernel(idx0_batch, idx1_batch, emb0_w, emb1_w, fc2_w, fc2_b, fc3_w,
           fc3_b, fc4_w, fc4_b):
    idx0 = idx0_batch.astype(jnp.int32)
    idx1 = idx1_batch.astype(jnp.int32)
    counts = _sc_hist(idx0, idx1)

    b2 = fc2_b.reshape(1, -1)
    b3 = fc3_b.reshape(1, -1)
    b4 = fc4_b.reshape(1, 1)
    out = pl.pallas_call(
        _tc_kernel,
        grid=(B // BT,),
        in_specs=[
            pl.BlockSpec((BT, CW), lambda i: (i, 0)),
            pl.BlockSpec((TABLE_SIZE, HIDDEN), lambda i: (0, 0)),
            pl.BlockSpec((TABLE_SIZE, HIDDEN), lambda i: (0, 0)),
            pl.BlockSpec(fc2_w.shape, lambda i: (0, 0)),
            pl.BlockSpec(b2.shape, lambda i: (0, 0)),
            pl.BlockSpec(fc3_w.shape, lambda i: (0, 0)),
            pl.BlockSpec(b3.shape, lambda i: (0, 0)),
            pl.BlockSpec(fc4_w.shape, lambda i: (0, 0)),
            pl.BlockSpec(b4.shape, lambda i: (0, 0)),
        ],
        out_specs=pl.BlockSpec((BT, 1), lambda i: (i, 0)),
        out_shape=jax.ShapeDtypeStruct((B, 1), jnp.float32),
        compiler_params=pltpu.CompilerParams(
            dimension_semantics=("arbitrary",),
        ),
    )(counts, emb0_w, emb1_w, fc2_w, b2, fc3_w, b3, fc4_w, b4)
    return out[:, 0]


# speedup vs baseline: 48.7654x; 1.2064x over previous
import functools

import jax
import jax.numpy as jnp
from jax import lax
from jax.experimental import pallas as pl
from jax.experimental.pallas import tpu as pltpu
from jax.experimental.pallas import tpu_sc as plsc

TABLE_SIZE = 640
HIDDEN = 256
B = 4096
L = 50

_info = plsc.get_sparse_core_info()
NC, NS = _info.num_cores, _info.num_subcores
NW = NC * NS
SPT = B // NW
QS = 32
NQ = SPT // QS
NG = QS // 16

_mesh = plsc.VectorSubcoreMesh(core_axis_name="c", subcore_axis_name="s")


@functools.partial(
    pl.kernel,
    mesh=_mesh,
    out_type=jax.ShapeDtypeStruct((B, TABLE_SIZE), jnp.int32),
    scratch_types=[
        pltpu.VMEM((L, SPT), jnp.int32),
        pltpu.VMEM((L, SPT), jnp.int32),
        pltpu.VMEM((QS, TABLE_SIZE), jnp.int32),
        pltpu.VMEM((QS, TABLE_SIZE), jnp.int32),
        pltpu.SemaphoreType.DMA,
        pltpu.SemaphoreType.DMA,
        pltpu.SemaphoreType.DMA,
    ],
    compiler_params=pltpu.CompilerParams(needs_layout_passes=False),
)
def _sc_hist(idx0_hbm, idx1_hbm, out_hbm, idx0_v, idx1_v, cnt_a, cnt_b,
             sem_a, sem_b, sem_i):
    wid = lax.axis_index("s") * NC + lax.axis_index("c")
    sbase = wid * SPT
    zero16 = jnp.zeros((16,), jnp.int32)
    lo16 = jnp.full((16,), 1, jnp.int32)
    hi16 = jnp.full((16,), 1 << 16, jnp.int32)
    lane = lax.iota(jnp.int32, 16)
    ld0 = pltpu.async_copy(idx0_hbm.at[:, pl.ds(sbase, SPT)], idx0_v, sem_i)
    ld1 = pltpu.async_copy(idx1_hbm.at[:, pl.ds(sbase, SPT)], idx1_v, sem_i)
    bufs = (cnt_a, cnt_b)
    sems = (sem_a, sem_b)
    pending = [None, None]
    for q in range(NQ):
        buf = bufs[q % 2]
        sem = sems[q % 2]
        if pending[q % 2] is not None:
            pending[q % 2].wait()

        def zrow_body(s, carry, buf=buf):
            for k in range(TABLE_SIZE // 16):
                buf[s, pl.ds(k * 16, 16)] = zero16
            return carry

        lax.fori_loop(0, QS, zrow_body, 0)
        if q == 0:
            ld0.wait()
            ld1.wait()

        for g in range(NG):
            col = q * QS + g * 16
            row16 = lane + (g * 16)

            def j_body(j, carry, buf=buf, col=col, row16=row16):
                cv0 = idx0_v[j, pl.ds(col, 16)]
                plsc.addupdate_scatter(buf, [row16, cv0], lo16)
                cv1 = idx1_v[j, pl.ds(col, 16)]
                plsc.addupdate_scatter(buf, [row16, cv1], hi16)
                return carry

            lax.fori_loop(0, L, j_body, 0)
        pending[q % 2] = pltpu.async_copy(
            buf, out_hbm.at[pl.ds(sbase + q * QS, QS)], sem)
    pending[0].wait()
    pending[1].wait()


BT = 512


def _tc_kernel(cnt_ref, emb0_ref, emb1_ref, w2_ref, b2_ref,
               w3_ref, b3_ref, w4_ref, b4_ref, out_ref):
    w = cnt_ref[...]
    c0 = jnp.bitwise_and(w, 0xFFFF).astype(jnp.float32)
    c1 = jnp.right_shift(w, 16).astype(jnp.float32)
    dn = (((1,), (0,)), ((), ()))
    sum0 = jax.lax.dot_general(c0, emb0_ref[...], dn,
                               preferred_element_type=jnp.float32)
    sum1 = jax.lax.dot_general(c1, emb1_ref[...], dn,
                               preferred_element_type=jnp.float32)
    h0 = jnp.maximum(sum0, 0.0)
    h1 = jnp.maximum(sum1, 0.0)
    dn_nt = (((1,), (1,)), ((), ()))
    w2 = w2_ref[...]
    x = (jax.lax.dot_general(h0, w2[:, :HIDDEN], dn_nt,
                             preferred_element_type=jnp.float32)
         + jax.lax.dot_general(h1, w2[:, HIDDEN:], dn_nt,
                               preferred_element_type=jnp.float32)
         + b2_ref[...])
    x = jnp.maximum(x, 0.0)
    x = jax.lax.dot_general(x, w3_ref[...], dn_nt,
                            preferred_element_type=jnp.float32) + b3_ref[...]
    x = jnp.maximum(x, 0.0)
    out = jnp.sum(x * w4_ref[...], axis=1, keepdims=True) + b4_ref[...]
    out_ref[...] = out


@jax.jit
def kernel(idx0_batch, idx1_batch, emb0_w, emb1_w, fc2_w, fc2_b, fc3_w,
           fc3_b, fc4_w, fc4_b):
    idx0_t = idx0_batch.astype(jnp.int32).T
    idx1_t = idx1_batch.astype(jnp.int32).T
    counts = _sc_hist(idx0_t, idx1_t)

    b2 = fc2_b.reshape(1, -1)
    b3 = fc3_b.reshape(1, -1)
    b4 = fc4_b.reshape(1, 1)
    out = pl.pallas_call(
        _tc_kernel,
        grid=(B // BT,),
        in_specs=[
            pl.BlockSpec((BT, TABLE_SIZE), lambda i: (i, 0)),
            pl.BlockSpec((TABLE_SIZE, HIDDEN), lambda i: (0, 0)),
            pl.BlockSpec((TABLE_SIZE, HIDDEN), lambda i: (0, 0)),
            pl.BlockSpec(fc2_w.shape, lambda i: (0, 0)),
            pl.BlockSpec(b2.shape, lambda i: (0, 0)),
            pl.BlockSpec(fc3_w.shape, lambda i: (0, 0)),
            pl.BlockSpec(b3.shape, lambda i: (0, 0)),
            pl.BlockSpec(fc4_w.shape, lambda i: (0, 0)),
            pl.BlockSpec(b4.shape, lambda i: (0, 0)),
        ],
        out_specs=pl.BlockSpec((BT, 1), lambda i: (i, 0)),
        out_shape=jax.ShapeDtypeStruct((B, 1), jnp.float32),
        compiler_params=pltpu.CompilerParams(
            dimension_semantics=("arbitrary",),
        ),
    )(counts, emb0_w, emb1_w, fc2_w, b2, fc3_w, b3, fc4_w, b4)
    return out[:, 0]


# speedup vs baseline: 51.1949x; 1.0498x over previous
import functools

import jax
import jax.numpy as jnp
from jax import lax
from jax.experimental import pallas as pl
from jax.experimental.pallas import tpu as pltpu
from jax.experimental.pallas import tpu_sc as plsc

TABLE_SIZE = 640
HIDDEN = 256
B = 4096
L = 50

_info = plsc.get_sparse_core_info()
NC, NS = _info.num_cores, _info.num_subcores
NW = NC * NS
SPT = B // NW
QS = 32
NQ = SPT // QS
NG = QS // 16

_mesh = plsc.VectorSubcoreMesh(core_axis_name="c", subcore_axis_name="s")


@functools.partial(
    pl.kernel,
    mesh=_mesh,
    out_type=jax.ShapeDtypeStruct((B, TABLE_SIZE), jnp.int32),
    scratch_types=[
        pltpu.VMEM((L, SPT), jnp.int32),
        pltpu.VMEM((L, SPT), jnp.int32),
        pltpu.VMEM((QS, TABLE_SIZE), jnp.int32),
        pltpu.VMEM((QS, TABLE_SIZE), jnp.int32),
        pltpu.SemaphoreType.DMA,
        pltpu.SemaphoreType.DMA,
        pltpu.SemaphoreType.DMA,
    ],
    compiler_params=pltpu.CompilerParams(needs_layout_passes=False),
)
def _sc_hist(idx0_hbm, idx1_hbm, out_hbm, idx0_v, idx1_v, cnt_a, cnt_b,
             sem_a, sem_b, sem_i):
    wid = lax.axis_index("s") * NC + lax.axis_index("c")
    sbase = wid * SPT
    zero16 = jnp.zeros((16,), jnp.int32)
    lo16 = jnp.full((16,), 1, jnp.int32)
    hi16 = jnp.full((16,), 1 << 16, jnp.int32)
    lane = lax.iota(jnp.int32, 16)
    ld0 = pltpu.async_copy(idx0_hbm.at[:, pl.ds(sbase, SPT)], idx0_v, sem_i)
    ld1 = pltpu.async_copy(idx1_hbm.at[:, pl.ds(sbase, SPT)], idx1_v, sem_i)
    bufs = (cnt_a, cnt_b)
    sems = (sem_a, sem_b)
    pending = [None, None]
    for q in range(NQ):
        buf = bufs[q % 2]
        sem = sems[q % 2]
        if pending[q % 2] is not None:
            pending[q % 2].wait()

        def zrow_body(s, carry, buf=buf):
            for k in range(TABLE_SIZE // 16):
                buf[s, pl.ds(k * 16, 16)] = zero16
            return carry

        lax.fori_loop(0, QS, zrow_body, 0)
        if q == 0:
            ld0.wait()
            ld1.wait()

        for g in range(NG):
            col = q * QS + g * 16
            row16 = lane + (g * 16)

            def j_body(j5, carry, buf=buf, col=col, row16=row16):
                for u in range(5):
                    j = j5 * 5 + u
                    cv0 = idx0_v[j, pl.ds(col, 16)]
                    plsc.addupdate_scatter(buf, [row16, cv0], lo16)
                    cv1 = idx1_v[j, pl.ds(col, 16)]
                    plsc.addupdate_scatter(buf, [row16, cv1], hi16)
                return carry

            lax.fori_loop(0, L // 5, j_body, 0)
        pending[q % 2] = pltpu.async_copy(
            buf, out_hbm.at[pl.ds(sbase + q * QS, QS)], sem)
    pending[0].wait()
    pending[1].wait()


BT = 512


def _tc_kernel(cnt_ref, emb0_ref, emb1_ref, w2_ref, b2_ref,
               w3_ref, b3_ref, w4_ref, b4_ref, out_ref):
    w = cnt_ref[...]
    c0 = jnp.bitwise_and(w, 0xFFFF).astype(jnp.float32)
    c1 = jnp.right_shift(w, 16).astype(jnp.float32)
    dn = (((1,), (0,)), ((), ()))
    sum0 = jax.lax.dot_general(c0, emb0_ref[...], dn,
                               preferred_element_type=jnp.float32)
    sum1 = jax.lax.dot_general(c1, emb1_ref[...], dn,
                               preferred_element_type=jnp.float32)
    h0 = jnp.maximum(sum0, 0.0)
    h1 = jnp.maximum(sum1, 0.0)
    dn_nt = (((1,), (1,)), ((), ()))
    w2 = w2_ref[...]
    x = (jax.lax.dot_general(h0, w2[:, :HIDDEN], dn_nt,
                             preferred_element_type=jnp.float32)
         + jax.lax.dot_general(h1, w2[:, HIDDEN:], dn_nt,
                               preferred_element_type=jnp.float32)
         + b2_ref[...])
    x = jnp.maximum(x, 0.0)
    x = jax.lax.dot_general(x, w3_ref[...], dn_nt,
                            preferred_element_type=jnp.float32) + b3_ref[...]
    x = jnp.maximum(x, 0.0)
    out_ref[...] = (jax.lax.dot_general(w4_ref[...], x, dn_nt,
                                        preferred_element_type=jnp.float32)
                    + b4_ref[0, 0])


@jax.jit
def kernel(idx0_batch, idx1_batch, emb0_w, emb1_w, fc2_w, fc2_b, fc3_w,
           fc3_b, fc4_w, fc4_b):
    idx0_t = idx0_batch.astype(jnp.int32).T
    idx1_t = idx1_batch.astype(jnp.int32).T
    counts = _sc_hist(idx0_t, idx1_t)

    b2 = fc2_b.reshape(1, -1)
    b3 = fc3_b.reshape(1, -1)
    b4 = fc4_b.reshape(1, 1)
    out = pl.pallas_call(
        _tc_kernel,
        grid=(B // BT,),
        in_specs=[
            pl.BlockSpec((BT, TABLE_SIZE), lambda i: (i, 0)),
            pl.BlockSpec((TABLE_SIZE, HIDDEN), lambda i: (0, 0)),
            pl.BlockSpec((TABLE_SIZE, HIDDEN), lambda i: (0, 0)),
            pl.BlockSpec(fc2_w.shape, lambda i: (0, 0)),
            pl.BlockSpec(b2.shape, lambda i: (0, 0)),
            pl.BlockSpec(fc3_w.shape, lambda i: (0, 0)),
            pl.BlockSpec(b3.shape, lambda i: (0, 0)),
            pl.BlockSpec(fc4_w.shape, lambda i: (0, 0)),
            pl.BlockSpec(b4.shape, lambda i: (0, 0)),
        ],
        out_specs=pl.BlockSpec((1, BT), lambda i: (0, i)),
        out_shape=jax.ShapeDtypeStruct((1, B), jnp.float32),
        compiler_params=pltpu.CompilerParams(
            dimension_semantics=("arbitrary",),
        ),
    )(counts, emb0_w, emb1_w, fc2_w, b2, fc3_w, b3, fc4_w, b4)
    return out[0]
